# Initial kernel scaffold; baseline (speedup 1.0000x reference)
import jax
import jax.numpy as jnp
from jax.experimental import pallas as pl


def kernel(kmer_ids, expected_signals, kmer_embed_high, kmer_embed_low, signal_proj_w, signal_proj_b, out_proj_w, out_proj_b, norm_w):
    raise NotImplementedError("write your pallas kernel here")


# speedup vs baseline: 2.1551x; 2.1551x over previous
import functools

import jax
import jax.numpy as jnp
from jax import lax
from jax.experimental import pallas as pl
from jax.experimental.pallas import tpu as pltpu
from jax.experimental.pallas import tpu_sc as plsc

D_MODEL = 128
HALF = 64
ESZ = 1000
N_TOK = 1024 * 200
NC, NS, LANES = 2, 16, 16
NW = NC * NS
TPW = N_TOK // NW
CHUNK = 128
NCHUNK = TPW // CHUNK
EPS = float(jnp.finfo(jnp.float32).eps)
TOK_TILE = 256


def _fold_body(eh, el, opw, spw, spb, opb, ehp, elp, u):
    w = opw[...]
    w1 = w[:, :HALF]
    w2 = w[:, HALF:]
    dn = (((1,), (1,)), ((), ()))
    c = lax.dot_general(spb[...], w2, dn,
                        preferred_element_type=jnp.float32) + opb[...]
    half_c = 0.5 * c
    ehp[...] = lax.dot_general(eh[...], w1, dn,
                               preferred_element_type=jnp.float32) + half_c
    elp[...] = lax.dot_general(el[...], w1, dn,
                               preferred_element_type=jnp.float32) + half_c
    u[...] = lax.dot_general(spw[...], w2, dn,
                             preferred_element_type=jnp.float32)


def _fold_tables(eh, el, opw, spw_row, spb_row, opb_row):
    return pl.pallas_call(
        _fold_body,
        out_shape=[
            jax.ShapeDtypeStruct((ESZ, D_MODEL), jnp.float32),
            jax.ShapeDtypeStruct((ESZ, D_MODEL), jnp.float32),
            jax.ShapeDtypeStruct((1, D_MODEL), jnp.float32),
        ],
    )(eh, el, opw, spw_row, spb_row, opb_row)


mesh = plsc.VectorSubcoreMesh(core_axis_name="c", subcore_axis_name="s")


@functools.partial(
    pl.kernel,
    out_type=jax.ShapeDtypeStruct((N_TOK, D_MODEL), jnp.float32),
    mesh=mesh,
    scratch_types=[
        pltpu.VMEM((CHUNK,), jnp.int32),
        pltpu.VMEM((CHUNK,), jnp.int32),
        pltpu.VMEM((CHUNK,), jnp.int32),
        pltpu.VMEM((CHUNK, D_MODEL), jnp.float32),
        pltpu.VMEM((CHUNK, D_MODEL), jnp.float32),
        pltpu.SemaphoreType.DMA,
        pltpu.SemaphoreType.DMA,
    ],
)
def _sc_gather(ids_hbm, ehp_hbm, elp_hbm, out_hbm,
               ids_v, idx_h, idx_l, rows_h, rows_l, sem_h, sem_l):
    wid = lax.axis_index("s") * NC + lax.axis_index("c")
    base0 = wid * TPW

    def chunk_body(ci, carry):
        base = base0 + ci * CHUNK
        pltpu.sync_copy(ids_hbm.at[pl.ds(base, CHUNK)], ids_v)
        esz_vec = jnp.full((LANES,), ESZ, dtype=jnp.int32)
        for i in range(CHUNK // LANES):
            sl = pl.ds(i * LANES, LANES)
            v = ids_v[sl]
            h = lax.div(v, esz_vec)
            idx_h[sl] = h
            idx_l[sl] = lax.sub(v, lax.mul(h, esz_vec))
        cp_h = pltpu.async_copy(ehp_hbm.at[idx_h], rows_h, sem_h)
        cp_l = pltpu.async_copy(elp_hbm.at[idx_l], rows_l, sem_l)
        cp_h.wait()
        cp_l.wait()

        def row_body(r, rcarry):
            for j in range(D_MODEL // LANES):
                sl = pl.ds(j * LANES, LANES)
                rows_h[r, sl] = rows_h[r, sl] + rows_l[r, sl]
            return rcarry

        lax.fori_loop(0, CHUNK, row_body, 0)
        pltpu.sync_copy(rows_h, out_hbm.at[pl.ds(base, CHUNK)])
        return carry

    lax.fori_loop(0, NCHUNK, chunk_body, 0)


def _norm_body(g, sig, u, nw, out):
    x = g[...] + sig[...] * u[...]
    ms = jnp.mean(x * x, axis=-1, keepdims=True)
    out[...] = x * lax.rsqrt(ms + EPS) * nw[...]


def _norm(g, sig2d, u_row, nw_row):
    grid = (N_TOK // TOK_TILE,)
    return pl.pallas_call(
        _norm_body,
        grid=grid,
        in_specs=[
            pl.BlockSpec((TOK_TILE, D_MODEL), lambda i: (i, 0)),
            pl.BlockSpec((TOK_TILE, 1), lambda i: (i, 0)),
            pl.BlockSpec((1, D_MODEL), lambda i: (0, 0)),
            pl.BlockSpec((1, D_MODEL), lambda i: (0, 0)),
        ],
        out_specs=pl.BlockSpec((TOK_TILE, D_MODEL), lambda i: (i, 0)),
        out_shape=jax.ShapeDtypeStruct((N_TOK, D_MODEL), jnp.float32),
    )(g, sig2d, u_row, nw_row)


def kernel(kmer_ids, expected_signals, kmer_embed_high, kmer_embed_low,
           signal_proj_w, signal_proj_b, out_proj_w, out_proj_b, norm_w):
    B, L = kmer_ids.shape
    ids = kmer_ids.reshape(-1).astype(jnp.int32)
    sig2d = expected_signals.reshape(-1, 1)
    ehp, elp, u_row = _fold_tables(
        kmer_embed_high, kmer_embed_low, out_proj_w,
        signal_proj_w.reshape(1, HALF), signal_proj_b.reshape(1, HALF),
        out_proj_b.reshape(1, D_MODEL))
    g = _sc_gather(ids, ehp, elp)
    out = _norm(g, sig2d, u_row, norm_w.reshape(1, D_MODEL))
    return out.reshape(B, L, D_MODEL)
